# baseline (speedup 1.0000x reference)
import functools

import jax
import jax.numpy as jnp
from jax import lax
from jax.experimental import pallas as pl
from jax.experimental.pallas import tpu as pltpu
from jax.experimental.pallas import tpu_sc as plsc

_C = 128
_K = 4
_G = _C * _K


@functools.lru_cache(maxsize=None)
def _build(n, v, d):
    info = plsc.get_sparse_core_info()
    nw = info.num_cores * info.num_subcores
    per_w = n // nw
    nch = per_w // _C
    ng = per_w // _G
    nt = ng // 2
    assert per_w % _G == 0 and ng % 2 == 0

    mesh = plsc.VectorSubcoreMesh(core_axis_name="c", subcore_axis_name="s")

    def body(table_hbm, idx_hbm, out_hbm, idx_v, buf_a, buf_b,
             gsem_a, gsem_b, osem_a, osem_b):
        wid = lax.axis_index("s") * info.num_cores + lax.axis_index("c")
        pltpu.sync_copy(idx_hbm.at[wid], idx_v)

        def fire_group(g, buf, sem):
            for b in range(_K):
                pltpu.async_copy(
                    table_hbm.at[idx_v.at[g * _K + b]],
                    buf.at[pl.ds(b * _C, _C)], sem)

        def wait_group(buf, sem):
            for b in range(_K):
                pltpu.make_async_copy(
                    table_hbm.at[idx_v.at[0]],
                    buf.at[pl.ds(b * _C, _C)], sem).wait()

        def wait_out(buf, sem):
            pltpu.make_async_copy(buf, out_hbm.at[0, 0], sem).wait()

        fire_group(0, buf_a, gsem_a)

        def step(t, carry):
            ge = 2 * t
            go = 2 * t + 1
            wait_group(buf_a, gsem_a)

            @pl.when(t > 0)
            def _():
                wait_out(buf_b, osem_b)

            fire_group(go, buf_b, gsem_b)
            pltpu.async_copy(buf_a, out_hbm.at[wid, ge], osem_a)
            wait_group(buf_b, gsem_b)
            wait_out(buf_a, osem_a)

            @pl.when(t < nt - 1)
            def _():
                fire_group(go + 1, buf_a, gsem_a)

            pltpu.async_copy(buf_b, out_hbm.at[wid, go], osem_b)
            return carry

        lax.fori_loop(0, nt, step, 0)
        wait_out(buf_b, osem_b)

    grid_kernel = pl.kernel(
        body,
        out_type=jax.ShapeDtypeStruct((nw, ng, _G, d), jnp.float32),
        mesh=mesh,
        scratch_types=(
            pltpu.VMEM((nch, _C), jnp.int32),
            pltpu.VMEM((_G, d), jnp.float32),
            pltpu.VMEM((_G, d), jnp.float32),
            pltpu.SemaphoreType.DMA,
            pltpu.SemaphoreType.DMA,
            pltpu.SemaphoreType.DMA,
            pltpu.SemaphoreType.DMA,
        ),
        compiler_params=pltpu.CompilerParams(use_tc_tiling_on_sc=False),
    )
    return grid_kernel, nw, ng


def kernel(input_ids, table):
    s, b = input_ids.shape
    v, d = table.shape
    n = s * b
    gather, nw, ng = _build(n, v, d)
    idx = input_ids.reshape(nw, n // (nw * _C), _C)
    out = gather(table, idx)
    return out.reshape(s, b, d)
